# Initial kernel scaffold; baseline (speedup 1.0000x reference)
import jax
import jax.numpy as jnp
from jax.experimental import pallas as pl


def kernel(edge_weight, user_table, faker_table, item_table, Wi1, bi1, Wi2, bi2, Wu1, bu1, Wu2, bu2, ln0_g, ln0_b, bn0_g, bn0_b, ln_g, ln_b, bn_g, bn_b, edge_index):
    raise NotImplementedError("write your pallas kernel here")


# speedup vs baseline: 2.1795x; 2.1795x over previous
import functools

import jax
import jax.numpy as jnp
from jax import lax
from jax.experimental import pallas as pl
from jax.experimental.pallas import tpu as pltpu
from jax.experimental.pallas import tpu_sc as plsc

_NU, _NF, _NI, _D = 6000, 500, 3500, 128
_N = _NU + _NF + _NI
_NUF = _NU + _NF
_E = 320000
_EPS = 1e-5

_NW = 32
_CSZ = 128
_CP = 82
_EPW = _CP * _CSZ
_EPAD = _NW * _EPW
_RPT = 624


_GDN = lax.GatherDimensionNumbers(offset_dims=(), collapsed_slice_dims=(0,),
                                  start_index_map=(0,))


def _lane_bcast(v16, lane):
    idx = jnp.full((16, 1), lane, jnp.int32)
    return lax.gather(v16, idx, _GDN, (1,),
                      mode=lax.GatherScatterMode.PROMISE_IN_BOUNDS)


def _sc_segment_sum(emb, pk, wf, zeros):
    mesh = plsc.VectorSubcoreMesh(core_axis_name="c", subcore_axis_name="s")

    @functools.partial(
        pl.kernel,
        mesh=mesh,
        out_type=jax.ShapeDtypeStruct((2, _N, _D), jnp.float32),
        scratch_types=[
            pltpu.VMEM((2, _CSZ), jnp.int32),
            pltpu.VMEM((2, _CSZ), jnp.int32),
            pltpu.VMEM((_CSZ,), jnp.float32),
            pltpu.VMEM((_CSZ,), jnp.float32),
            pltpu.VMEM((_CSZ, _D), jnp.float32),
            pltpu.VMEM((_CSZ, _D), jnp.float32),
            pltpu.VMEM_SHARED((_N, _D), jnp.float32),
            pltpu.SemaphoreType.DMA,
            pltpu.SemaphoreType.DMA,
        ],
    )
    def k(emb_h, pk_h, wf_h, z_h, out_h, pk0, pk1, wv0, wv1, buf0, buf1,
          acc, sem0, sem1):
        c = lax.axis_index("c")
        s = lax.axis_index("s")
        wid = s * 2 + c
        bufs = (buf0, buf1)
        pks = (pk0, pk1)
        wvs = (wv0, wv1)
        sems = (sem0, sem1)

        base = pl.multiple_of(s * _RPT, 8)
        pltpu.sync_copy(z_h.at[pl.ds(base, _RPT)],
                        acc.at[pl.ds(base, _RPT)])

        @pl.when(s == 15)
        def _():
            pltpu.sync_copy(z_h.at[pl.ds(16 * _RPT, _N - 16 * _RPT)],
                            acc.at[pl.ds(16 * _RPT, _N - 16 * _RPT)])

        plsc.subcore_barrier()

        for b in range(2):
            pltpu.sync_copy(pk_h.at[wid, b], pks[b])
            pltpu.sync_copy(wf_h.at[wid, b], wvs[b])
            pltpu.async_copy(emb_h.at[pks[b].at[0]], bufs[b], sems[b])

        def pair(i, _):
            for b in range(2):
                ci = 2 * i + b
                pltpu.make_async_copy(emb_h.at[pks[b].at[0]], bufs[b],
                                      sems[b]).wait()

                def group(g, _2):
                    w16 = wvs[b][pl.ds(pl.multiple_of(g * 16, 16), 16)]
                    for e16 in range(16):
                        wv = _lane_bcast(w16, e16)
                        e = g * 16 + e16
                        for j in range(8):
                            sl = pl.ds(j * 16, 16)
                            bufs[b][e, sl] = bufs[b][e, sl] * wv
                    return 0

                lax.fori_loop(0, _CSZ // 16, group, 0)
                pltpu.sync_copy(bufs[b], acc.at[pks[b].at[1]], add=True)

                @pl.when(i < _CP // 2 - 1)
                def _():
                    pltpu.sync_copy(pk_h.at[wid, ci + 2], pks[b])
                    pltpu.sync_copy(wf_h.at[wid, ci + 2], wvs[b])
                    pltpu.async_copy(emb_h.at[pks[b].at[0]], bufs[b],
                                     sems[b])
            return 0

        lax.fori_loop(0, _CP // 2, pair, 0)
        plsc.subcore_barrier()
        pltpu.sync_copy(acc.at[pl.ds(base, _RPT)],
                        out_h.at[c, pl.ds(base, _RPT)])

        @pl.when(s == 15)
        def _():
            pltpu.sync_copy(acc.at[pl.ds(16 * _RPT, _N - 16 * _RPT)],
                            out_h.at[c, pl.ds(16 * _RPT, _N - 16 * _RPT)])

    return k(emb, pk, wf, zeros)


def _tc_prologue(ue, it, Wu1, bu1, Wu2, bu2, Wi1, bi1, Wi2, bi2,
                 g0, b0, gb0, bb0):
    def body(ue_r, it_r, wu1, bu1r, wu2, bu2r, wi1, bi1r, wi2, bi2r,
             g0r, b0r, gb0r, bb0r, out_r):
        f32 = jnp.float32
        u = ue_r[...]
        h = jnp.maximum(jnp.dot(u, wu1[...], preferred_element_type=f32)
                        + bu1r[...], 0.0)
        u2 = jnp.dot(h, wu2[...], preferred_element_type=f32) + bu2r[...]
        t = it_r[...]
        h2 = jnp.maximum(jnp.dot(t, wi1[...], preferred_element_type=f32)
                         + bi1r[...], 0.0)
        t2 = jnp.dot(h2, wi2[...], preferred_element_type=f32) + bi2r[...]
        x = jnp.concatenate([u2, t2], axis=0)
        mu = jnp.mean(x, axis=1, keepdims=True)
        var = jnp.mean((x - mu) ** 2, axis=1, keepdims=True)
        x = (x - mu) / jnp.sqrt(var + _EPS) * g0r[...] + b0r[...]
        mu0 = jnp.mean(x, axis=0, keepdims=True)
        var0 = jnp.mean((x - mu0) ** 2, axis=0, keepdims=True)
        out_r[...] = (x - mu0) / jnp.sqrt(var0 + _EPS) * gb0r[...] + bb0r[...]

    return pl.pallas_call(
        body,
        out_shape=jax.ShapeDtypeStruct((_N, _D), jnp.float32),
    )(ue, it, Wu1, bu1, Wu2, bu2, Wi1, bi1, Wi2, bi2, g0, b0, gb0, bb0)


def _tc_layer(agg, ori, g, b, gb, bb, relu):
    def body(agg_r, ori_r, g_r, b_r, gb_r, bb_r, out_r):
        x = ori_r[...] + agg_r[0] + agg_r[1]
        mu = jnp.mean(x, axis=1, keepdims=True)
        var = jnp.mean((x - mu) ** 2, axis=1, keepdims=True)
        x = (x - mu) / jnp.sqrt(var + _EPS) * g_r[...] + b_r[...]
        mu0 = jnp.mean(x, axis=0, keepdims=True)
        var0 = jnp.mean((x - mu0) ** 2, axis=0, keepdims=True)
        x = (x - mu0) / jnp.sqrt(var0 + _EPS) * gb_r[...] + bb_r[...]
        if relu:
            x = jnp.maximum(x, 0.0)
        out_r[...] = x

    return pl.pallas_call(
        body,
        out_shape=jax.ShapeDtypeStruct((_N, _D), jnp.float32),
    )(agg, ori, g, b, gb, bb)


def kernel(edge_weight, user_table, faker_table, item_table, Wi1, bi1, Wi2,
           bi2, Wu1, bu1, Wu2, bu2, ln0_g, ln0_b, bn0_g, bn0_b, ln_g, ln_b,
           bn_g, bn_b, edge_index):
    ue = jnp.concatenate([user_table, faker_table], axis=0)
    r = lambda v: v.reshape(1, _D)

    pad = _EPAD - _E
    src3 = jnp.pad(edge_index[1], (0, pad)).astype(jnp.int32)
    src3 = src3.reshape(_NW, _CP, _CSZ)
    dst3 = jnp.pad(edge_index[0], (0, pad)).astype(jnp.int32)
    dst3 = dst3.reshape(_NW, _CP, _CSZ)
    wf = jnp.pad(edge_weight, (0, pad)).reshape(_NW, _CP, _CSZ)
    pk = jnp.stack([src3, dst3], axis=2)
    zeros = jnp.zeros((_N, _D), jnp.float32)

    embs_ori = _tc_prologue(ue, item_table, Wu1, r(bu1), Wu2, r(bu2),
                            Wi1, r(bi1), Wi2, r(bi2),
                            r(ln0_g), r(ln0_b), r(bn0_g), r(bn0_b))
    x = embs_ori
    for layer in range(3):
        agg = _sc_segment_sum(x, pk, wf, zeros)
        x = _tc_layer(agg, embs_ori, r(ln_g[layer]), r(ln_b[layer]),
                      r(bn_g[layer]), r(bn_b[layer]), relu=layer != 2)
    return x[:_NUF], x[_NUF:]


# speedup vs baseline: 2.1812x; 1.0008x over previous
import functools

import jax
import jax.numpy as jnp
from jax import lax
from jax.experimental import pallas as pl
from jax.experimental.pallas import tpu as pltpu
from jax.experimental.pallas import tpu_sc as plsc

_NU, _NF, _NI, _D = 6000, 500, 3500, 128
_N = _NU + _NF + _NI
_NUF = _NU + _NF
_E = 320000
_EPS = 1e-5

_NW = 32
_CSZ = 128
_CP = 82
_EPW = _CP * _CSZ
_EPAD = _NW * _EPW
_RPT = 624


_GDN = lax.GatherDimensionNumbers(offset_dims=(), collapsed_slice_dims=(0,),
                                  start_index_map=(0,))


def _lane_bcast(v16, lane):
    idx = jnp.full((16, 1), lane, jnp.int32)
    return lax.gather(v16, idx, _GDN, (1,),
                      mode=lax.GatherScatterMode.PROMISE_IN_BOUNDS)


def _sc_segment_sum(emb, pk, wf, zeros):
    mesh = plsc.VectorSubcoreMesh(core_axis_name="c", subcore_axis_name="s")

    @functools.partial(
        pl.kernel,
        mesh=mesh,
        out_type=jax.ShapeDtypeStruct((2, _N, _D), jnp.float32),
        scratch_types=[
            pltpu.VMEM((2, _CSZ), jnp.int32),
            pltpu.VMEM((2, _CSZ), jnp.int32),
            pltpu.VMEM((_CSZ,), jnp.float32),
            pltpu.VMEM((_CSZ,), jnp.float32),
            pltpu.VMEM((_CSZ, _D), jnp.float32),
            pltpu.VMEM((_CSZ, _D), jnp.float32),
            pltpu.VMEM_SHARED((_N, _D), jnp.float32),
            pltpu.SemaphoreType.DMA,
            pltpu.SemaphoreType.DMA,
        ],
    )
    def k(emb_h, pk_h, wf_h, z_h, out_h, pk0, pk1, wv0, wv1, buf0, buf1,
          acc, sem0, sem1):
        c = lax.axis_index("c")
        s = lax.axis_index("s")
        wid = s * 2 + c
        bufs = (buf0, buf1)
        pks = (pk0, pk1)
        wvs = (wv0, wv1)
        sems = (sem0, sem1)

        base = pl.multiple_of(s * _RPT, 8)
        pltpu.sync_copy(z_h.at[pl.ds(base, _RPT)],
                        acc.at[pl.ds(base, _RPT)])

        @pl.when(s == 15)
        def _():
            pltpu.sync_copy(z_h.at[pl.ds(16 * _RPT, _N - 16 * _RPT)],
                            acc.at[pl.ds(16 * _RPT, _N - 16 * _RPT)])

        plsc.subcore_barrier()

        def start_gather(b):
            for q in range(4):
                pltpu.async_copy(
                    emb_h.at[pks[b].at[0, pl.ds(q * 32, 32)]],
                    bufs[b].at[pl.ds(q * 32, 32)], sems[b])

        def wait_gather(b):
            for q in range(4):
                pltpu.make_async_copy(
                    emb_h.at[pks[b].at[0, pl.ds(q * 32, 32)]],
                    bufs[b].at[pl.ds(q * 32, 32)], sems[b]).wait()

        for b in range(2):
            pltpu.sync_copy(pk_h.at[wid, b], pks[b])
            pltpu.sync_copy(wf_h.at[wid, b], wvs[b])
            start_gather(b)

        def pair(i, _):
            for b in range(2):
                ci = 2 * i + b
                wait_gather(b)

                def group(g, _2):
                    w16 = wvs[b][pl.ds(pl.multiple_of(g * 16, 16), 16)]
                    for e16 in range(16):
                        wv = _lane_bcast(w16, e16)
                        e = g * 16 + e16
                        for j in range(8):
                            sl = pl.ds(j * 16, 16)
                            bufs[b][e, sl] = bufs[b][e, sl] * wv
                    return 0

                lax.fori_loop(0, _CSZ // 16, group, 0)
                pltpu.sync_copy(bufs[b], acc.at[pks[b].at[1]], add=True)

                @pl.when(i < _CP // 2 - 1)
                def _():
                    pltpu.sync_copy(pk_h.at[wid, ci + 2], pks[b])
                    pltpu.sync_copy(wf_h.at[wid, ci + 2], wvs[b])
                    start_gather(b)
            return 0

        lax.fori_loop(0, _CP // 2, pair, 0)
        plsc.subcore_barrier()
        pltpu.sync_copy(acc.at[pl.ds(base, _RPT)],
                        out_h.at[c, pl.ds(base, _RPT)])

        @pl.when(s == 15)
        def _():
            pltpu.sync_copy(acc.at[pl.ds(16 * _RPT, _N - 16 * _RPT)],
                            out_h.at[c, pl.ds(16 * _RPT, _N - 16 * _RPT)])

    return k(emb, pk, wf, zeros)


def _tc_prologue(ue, it, Wu1, bu1, Wu2, bu2, Wi1, bi1, Wi2, bi2,
                 g0, b0, gb0, bb0):
    def body(ue_r, it_r, wu1, bu1r, wu2, bu2r, wi1, bi1r, wi2, bi2r,
             g0r, b0r, gb0r, bb0r, out_r):
        f32 = jnp.float32
        u = ue_r[...]
        h = jnp.maximum(jnp.dot(u, wu1[...], preferred_element_type=f32)
                        + bu1r[...], 0.0)
        u2 = jnp.dot(h, wu2[...], preferred_element_type=f32) + bu2r[...]
        t = it_r[...]
        h2 = jnp.maximum(jnp.dot(t, wi1[...], preferred_element_type=f32)
                         + bi1r[...], 0.0)
        t2 = jnp.dot(h2, wi2[...], preferred_element_type=f32) + bi2r[...]
        x = jnp.concatenate([u2, t2], axis=0)
        mu = jnp.mean(x, axis=1, keepdims=True)
        var = jnp.mean((x - mu) ** 2, axis=1, keepdims=True)
        x = (x - mu) / jnp.sqrt(var + _EPS) * g0r[...] + b0r[...]
        mu0 = jnp.mean(x, axis=0, keepdims=True)
        var0 = jnp.mean((x - mu0) ** 2, axis=0, keepdims=True)
        out_r[...] = (x - mu0) / jnp.sqrt(var0 + _EPS) * gb0r[...] + bb0r[...]

    return pl.pallas_call(
        body,
        out_shape=jax.ShapeDtypeStruct((_N, _D), jnp.float32),
    )(ue, it, Wu1, bu1, Wu2, bu2, Wi1, bi1, Wi2, bi2, g0, b0, gb0, bb0)


def _tc_layer(agg, ori, g, b, gb, bb, relu):
    def body(agg_r, ori_r, g_r, b_r, gb_r, bb_r, out_r):
        x = ori_r[...] + agg_r[0] + agg_r[1]
        mu = jnp.mean(x, axis=1, keepdims=True)
        var = jnp.mean((x - mu) ** 2, axis=1, keepdims=True)
        x = (x - mu) / jnp.sqrt(var + _EPS) * g_r[...] + b_r[...]
        mu0 = jnp.mean(x, axis=0, keepdims=True)
        var0 = jnp.mean((x - mu0) ** 2, axis=0, keepdims=True)
        x = (x - mu0) / jnp.sqrt(var0 + _EPS) * gb_r[...] + bb_r[...]
        if relu:
            x = jnp.maximum(x, 0.0)
        out_r[...] = x

    return pl.pallas_call(
        body,
        out_shape=jax.ShapeDtypeStruct((_N, _D), jnp.float32),
    )(agg, ori, g, b, gb, bb)


def kernel(edge_weight, user_table, faker_table, item_table, Wi1, bi1, Wi2,
           bi2, Wu1, bu1, Wu2, bu2, ln0_g, ln0_b, bn0_g, bn0_b, ln_g, ln_b,
           bn_g, bn_b, edge_index):
    ue = jnp.concatenate([user_table, faker_table], axis=0)
    r = lambda v: v.reshape(1, _D)

    pad = _EPAD - _E
    src3 = jnp.pad(edge_index[1], (0, pad)).astype(jnp.int32)
    src3 = src3.reshape(_NW, _CP, _CSZ)
    dst3 = jnp.pad(edge_index[0], (0, pad)).astype(jnp.int32)
    dst3 = dst3.reshape(_NW, _CP, _CSZ)
    wf = jnp.pad(edge_weight, (0, pad)).reshape(_NW, _CP, _CSZ)
    pk = jnp.stack([src3, dst3], axis=2)
    zeros = jnp.zeros((_N, _D), jnp.float32)

    embs_ori = _tc_prologue(ue, item_table, Wu1, r(bu1), Wu2, r(bu2),
                            Wi1, r(bi1), Wi2, r(bi2),
                            r(ln0_g), r(ln0_b), r(bn0_g), r(bn0_b))
    x = embs_ori
    for layer in range(3):
        agg = _sc_segment_sum(x, pk, wf, zeros)
        x = _tc_layer(agg, embs_ori, r(ln_g[layer]), r(ln_b[layer]),
                      r(bn_g[layer]), r(bn_b[layer]), relu=layer != 2)
    return x[:_NUF], x[_NUF:]


# speedup vs baseline: 6.0473x; 2.7725x over previous
import functools

import jax
import jax.numpy as jnp
from jax import lax
from jax.experimental import pallas as pl
from jax.experimental.pallas import tpu as pltpu
from jax.experimental.pallas import tpu_sc as plsc

_NU, _NF, _NI, _D = 6000, 500, 3500, 128
_N = _NU + _NF + _NI
_NUF = _NU + _NF
_E = 320000
_EPS = 1e-5

_NW = 32
_CSZ = 128
_CP = 82
_EPW = _CP * _CSZ
_EPAD = _NW * _EPW
_RPT = 624

_GDN = lax.GatherDimensionNumbers(offset_dims=(), collapsed_slice_dims=(0,),
                                  start_index_map=(0,))
_MESH = plsc.VectorSubcoreMesh(core_axis_name="c", subcore_axis_name="s")


def _lane_bcast(v16, lane):
    idx = jnp.full((16, 1), lane, jnp.int32)
    return lax.gather(v16, idx, _GDN, (1,),
                      mode=lax.GatherScatterMode.PROMISE_IN_BOUNDS)


def _stage_rows(src_ref, dst_ref, s):
    base = pl.multiple_of(s * _RPT, 8)
    pltpu.sync_copy(src_ref.at[pl.ds(base, _RPT)],
                    dst_ref.at[pl.ds(base, _RPT)])

    @pl.when(s == 15)
    def _():
        pltpu.sync_copy(src_ref.at[pl.ds(16 * _RPT, _N - 16 * _RPT)],
                        dst_ref.at[pl.ds(16 * _RPT, _N - 16 * _RPT)])


def _sc_gather_scale(emb, src3, w3):
    @functools.partial(
        pl.kernel,
        mesh=_MESH,
        out_type=jax.ShapeDtypeStruct((_NW, _EPW, _D), jnp.float32),
        scratch_types=[
            pltpu.VMEM((_CP, _CSZ), jnp.int32),
            pltpu.VMEM((_CSZ,), jnp.float32),
            pltpu.VMEM((_CSZ,), jnp.float32),
            pltpu.VMEM((_CSZ, _D), jnp.float32),
            pltpu.VMEM((_CSZ, _D), jnp.float32),
            pltpu.VMEM_SHARED((_N, _D), jnp.float32),
            pltpu.SemaphoreType.DMA,
            pltpu.SemaphoreType.DMA,
            pltpu.SemaphoreType.DMA,
            pltpu.SemaphoreType.DMA,
        ],
    )
    def k(emb_h, src_h, w_h, msg_h, src_v, wv0, wv1, buf0, buf1, emb_s,
          wsem0, wsem1, msem0, msem1):
        c = lax.axis_index("c")
        s = lax.axis_index("s")
        wid = s * 2 + c
        bufs = (buf0, buf1)
        wvs = (wv0, wv1)
        wsems = (wsem0, wsem1)
        msems = (msem0, msem1)

        def msg_slice(ci):
            return msg_h.at[wid, pl.ds(pl.multiple_of(ci * _CSZ, 128), _CSZ)]

        _stage_rows(emb_h, emb_s, s)
        pltpu.sync_copy(src_h.at[wid], src_v)
        for b in range(2):
            pltpu.sync_copy(w_h.at[wid, b], wvs[b])
        plsc.subcore_barrier()

        def pair(i, _):
            for b in range(2):
                ci = 2 * i + b

                @pl.when(i > 0)
                def _():
                    pltpu.make_async_copy(bufs[b], msg_slice(ci - 2),
                                          msems[b]).wait()
                    pltpu.make_async_copy(w_h.at[wid, ci], wvs[b],
                                          wsems[b]).wait()

                pltpu.sync_copy(emb_s.at[src_v.at[ci]], bufs[b])

                def group(g, _2):
                    w16 = wvs[b][pl.ds(pl.multiple_of(g * 16, 16), 16)]
                    for e16 in range(16):
                        wv = _lane_bcast(w16, e16)
                        e = g * 16 + e16
                        for j in range(8):
                            sl = pl.ds(j * 16, 16)
                            bufs[b][e, sl] = bufs[b][e, sl] * wv
                    return 0

                lax.fori_loop(0, _CSZ // 16, group, 0)
                pltpu.async_copy(bufs[b], msg_slice(ci), msems[b])

                @pl.when(i < _CP // 2 - 1)
                def _():
                    pltpu.async_copy(w_h.at[wid, ci + 2], wvs[b], wsems[b])
            return 0

        lax.fori_loop(0, _CP // 2, pair, 0)
        for b in range(2):
            pltpu.make_async_copy(bufs[b], msg_slice(_CP - 2 + b),
                                  msems[b]).wait()

    return k(emb, src3, w3)


def _sc_scatter_add(msg, dst3, zeros):
    @functools.partial(
        pl.kernel,
        mesh=_MESH,
        out_type=jax.ShapeDtypeStruct((2, _N, _D), jnp.float32),
        scratch_types=[
            pltpu.VMEM((_CP, _CSZ), jnp.int32),
            pltpu.VMEM((_CSZ, _D), jnp.float32),
            pltpu.VMEM((_CSZ, _D), jnp.float32),
            pltpu.VMEM_SHARED((_N, _D), jnp.float32),
            pltpu.SemaphoreType.DMA,
            pltpu.SemaphoreType.DMA,
        ],
    )
    def k(msg_h, dst_h, z_h, out_h, dst_v, buf0, buf1, acc, sem0, sem1):
        c = lax.axis_index("c")
        s = lax.axis_index("s")
        wid = s * 2 + c
        bufs = (buf0, buf1)
        sems = (sem0, sem1)

        def msg_slice(ci):
            return msg_h.at[wid, pl.ds(pl.multiple_of(ci * _CSZ, 128), _CSZ)]

        _stage_rows(z_h, acc, s)
        pltpu.sync_copy(dst_h.at[wid], dst_v)
        plsc.subcore_barrier()

        for b in range(2):
            pltpu.async_copy(msg_slice(b), bufs[b], sems[b])

        def pair(i, _):
            for b in range(2):
                ci = 2 * i + b
                pltpu.make_async_copy(msg_slice(ci), bufs[b], sems[b]).wait()
                pltpu.sync_copy(bufs[b], acc.at[dst_v.at[ci]], add=True)

                @pl.when(i < _CP // 2 - 1)
                def _():
                    pltpu.async_copy(msg_slice(ci + 2), bufs[b], sems[b])
            return 0

        lax.fori_loop(0, _CP // 2, pair, 0)
        plsc.subcore_barrier()
        _stage_rows(acc, out_h.at[c], s)

    return k(msg, dst3, zeros)


def _tc_prologue(ue, it, Wu1, bu1, Wu2, bu2, Wi1, bi1, Wi2, bi2,
                 g0, b0, gb0, bb0):
    def body(ue_r, it_r, wu1, bu1r, wu2, bu2r, wi1, bi1r, wi2, bi2r,
             g0r, b0r, gb0r, bb0r, out_r):
        f32 = jnp.float32
        u = ue_r[...]
        h = jnp.maximum(jnp.dot(u, wu1[...], preferred_element_type=f32)
                        + bu1r[...], 0.0)
        u2 = jnp.dot(h, wu2[...], preferred_element_type=f32) + bu2r[...]
        t = it_r[...]
        h2 = jnp.maximum(jnp.dot(t, wi1[...], preferred_element_type=f32)
                         + bi1r[...], 0.0)
        t2 = jnp.dot(h2, wi2[...], preferred_element_type=f32) + bi2r[...]
        x = jnp.concatenate([u2, t2], axis=0)
        mu = jnp.mean(x, axis=1, keepdims=True)
        var = jnp.mean((x - mu) ** 2, axis=1, keepdims=True)
        x = (x - mu) / jnp.sqrt(var + _EPS) * g0r[...] + b0r[...]
        mu0 = jnp.mean(x, axis=0, keepdims=True)
        var0 = jnp.mean((x - mu0) ** 2, axis=0, keepdims=True)
        out_r[...] = (x - mu0) / jnp.sqrt(var0 + _EPS) * gb0r[...] + bb0r[...]

    return pl.pallas_call(
        body,
        out_shape=jax.ShapeDtypeStruct((_N, _D), jnp.float32),
    )(ue, it, Wu1, bu1, Wu2, bu2, Wi1, bi1, Wi2, bi2, g0, b0, gb0, bb0)


def _tc_layer(agg, ori, g, b, gb, bb, relu):
    def body(agg_r, ori_r, g_r, b_r, gb_r, bb_r, out_r):
        x = ori_r[...] + agg_r[0] + agg_r[1]
        mu = jnp.mean(x, axis=1, keepdims=True)
        var = jnp.mean((x - mu) ** 2, axis=1, keepdims=True)
        x = (x - mu) / jnp.sqrt(var + _EPS) * g_r[...] + b_r[...]
        mu0 = jnp.mean(x, axis=0, keepdims=True)
        var0 = jnp.mean((x - mu0) ** 2, axis=0, keepdims=True)
        x = (x - mu0) / jnp.sqrt(var0 + _EPS) * gb_r[...] + bb_r[...]
        if relu:
            x = jnp.maximum(x, 0.0)
        out_r[...] = x

    return pl.pallas_call(
        body,
        out_shape=jax.ShapeDtypeStruct((_N, _D), jnp.float32),
    )(agg, ori, g, b, gb, bb)


def kernel(edge_weight, user_table, faker_table, item_table, Wi1, bi1, Wi2,
           bi2, Wu1, bu1, Wu2, bu2, ln0_g, ln0_b, bn0_g, bn0_b, ln_g, ln_b,
           bn_g, bn_b, edge_index):
    ue = jnp.concatenate([user_table, faker_table], axis=0)
    r = lambda v: v.reshape(1, _D)

    pad = _EPAD - _E
    src3 = jnp.pad(edge_index[1], (0, pad)).astype(jnp.int32)
    src3 = src3.reshape(_NW, _CP, _CSZ)
    dst3 = jnp.pad(edge_index[0], (0, pad)).astype(jnp.int32)
    dst3 = dst3.reshape(_NW, _CP, _CSZ)
    w3 = jnp.pad(edge_weight, (0, pad)).reshape(_NW, _CP, _CSZ)
    zeros = jnp.zeros((_N, _D), jnp.float32)

    embs_ori = _tc_prologue(ue, item_table, Wu1, r(bu1), Wu2, r(bu2),
                            Wi1, r(bi1), Wi2, r(bi2),
                            r(ln0_g), r(ln0_b), r(bn0_g), r(bn0_b))
    x = embs_ori
    for layer in range(3):
        msg = _sc_gather_scale(x, src3, w3)
        agg = _sc_scatter_add(msg, dst3, zeros)
        x = _tc_layer(agg, embs_ori, r(ln_g[layer]), r(ln_b[layer]),
                      r(bn_g[layer]), r(bn_b[layer]), relu=layer != 2)
    return x[:_NUF], x[_NUF:]


# speedup vs baseline: 6.4911x; 1.0734x over previous
import functools

import jax
import jax.numpy as jnp
from jax import lax
from jax.experimental import pallas as pl
from jax.experimental.pallas import tpu as pltpu
from jax.experimental.pallas import tpu_sc as plsc

_NU, _NF, _NI, _D = 6000, 500, 3500, 128
_N = _NU + _NF + _NI
_NUF = _NU + _NF
_E = 320000
_EPS = 1e-5

_NW = 32
_CSZ = 64
_CP = 168
_EPW = _CP * _CSZ
_EPAD = _NW * _EPW
_RPT = 624

_GDN = lax.GatherDimensionNumbers(offset_dims=(), collapsed_slice_dims=(0,),
                                  start_index_map=(0,))
_MESH = plsc.VectorSubcoreMesh(core_axis_name="c", subcore_axis_name="s")


def _lane_bcast(v16, lane):
    idx = jnp.full((16, 1), lane, jnp.int32)
    return lax.gather(v16, idx, _GDN, (1,),
                      mode=lax.GatherScatterMode.PROMISE_IN_BOUNDS)


def _stage_rows(src_ref, dst_ref, s):
    base = pl.multiple_of(s * _RPT, 8)
    pltpu.sync_copy(src_ref.at[pl.ds(base, _RPT)],
                    dst_ref.at[pl.ds(base, _RPT)])

    @pl.when(s == 15)
    def _():
        pltpu.sync_copy(src_ref.at[pl.ds(16 * _RPT, _N - 16 * _RPT)],
                        dst_ref.at[pl.ds(16 * _RPT, _N - 16 * _RPT)])


def _sc_gather_scale(emb, src4, w4):
    @functools.partial(
        pl.kernel,
        mesh=_MESH,
        out_type=jax.ShapeDtypeStruct((_NW, _EPW, _D), jnp.float32),
        scratch_types=[
            pltpu.VMEM((_CP // 2, 2 * _CSZ), jnp.int32),
            pltpu.VMEM((_EPW,), jnp.float32),
            pltpu.VMEM((_CSZ, _D), jnp.float32),
            pltpu.VMEM((_CSZ, _D), jnp.float32),
            pltpu.VMEM((_CSZ, _D), jnp.float32),
            pltpu.VMEM_SHARED((_N, _D), jnp.float32),
            pltpu.SemaphoreType.DMA,
            pltpu.SemaphoreType.DMA,
            pltpu.SemaphoreType.DMA,
            pltpu.SemaphoreType.DMA,
            pltpu.SemaphoreType.DMA,
            pltpu.SemaphoreType.DMA,
        ],
    )
    def k(emb_h, src_h, w_h, msg_h, src_v, w_v, buf0, buf1, buf2, emb_s,
          gsem0, gsem1, gsem2, msem0, msem1, msem2):
        c = lax.axis_index("c")
        s = lax.axis_index("s")
        wid = s * 2 + c
        bufs = (buf0, buf1, buf2)
        gsems = (gsem0, gsem1, gsem2)
        msems = (msem0, msem1, msem2)

        def msg_slice(ci):
            return msg_h.at[wid, pl.ds(pl.multiple_of(ci * _CSZ, 64), _CSZ)]

        def src_ref(ci):
            return src_v.at[ci // 2,
                            pl.ds(pl.multiple_of((ci % 2) * _CSZ, 64), _CSZ)]

        def issue_gather(ci, r):
            pltpu.async_copy(emb_s.at[src_ref(ci)], bufs[r], gsems[r])

        def wait_gather(ci, r):
            pltpu.make_async_copy(emb_s.at[src_ref(ci)], bufs[r],
                                  gsems[r]).wait()

        def issue_write(ci, r):
            pltpu.async_copy(bufs[r], msg_slice(ci), msems[r])

        def wait_write(ci, r):
            pltpu.make_async_copy(bufs[r], msg_slice(ci), msems[r]).wait()

        def scale(ci, r):
            def group(g, _2):
                off = pl.multiple_of(ci * _CSZ + g * 16, 16)
                w16 = w_v[pl.ds(off, 16)]
                for e16 in range(16):
                    wv = _lane_bcast(w16, e16)
                    e = g * 16 + e16
                    for j in range(8):
                        sl = pl.ds(j * 16, 16)
                        bufs[r][e, sl] = bufs[r][e, sl] * wv
                return 0

            lax.fori_loop(0, _CSZ // 16, group, 0)

        _stage_rows(emb_h, emb_s, s)
        pltpu.sync_copy(src_h.at[wid], src_v)
        pltpu.sync_copy(w_h.at[wid], w_v)
        plsc.subcore_barrier()

        issue_gather(0, 0)
        issue_gather(1, 1)
        for ci in range(3):
            wait_gather(ci, ci)
            scale(ci, ci)
            issue_write(ci, ci)
            if ci >= 1:
                wait_write(ci - 1, ci - 1)
            issue_gather(ci + 2, (ci + 2) % 3)

        def body(i, _):
            for r in range(3):
                ci = 3 * i + r
                wait_gather(ci, r)
                scale(ci, r)
                issue_write(ci, r)

                @pl.when(ci + 2 < _CP)
                def _():
                    wait_write(ci - 1, (r - 1) % 3)
                    issue_gather(ci + 2, (r + 2) % 3)
            return 0

        lax.fori_loop(1, _CP // 3, body, 0)
        for ci in range(_CP - 3, _CP):
            wait_write(ci, ci % 3)

    return k(emb, src4, w4)


def _sc_scatter_add(msg, dst4, zeros):
    @functools.partial(
        pl.kernel,
        mesh=_MESH,
        out_type=jax.ShapeDtypeStruct((2, _N, _D), jnp.float32),
        scratch_types=[
            pltpu.VMEM((_CP, _CSZ), jnp.int32),
            pltpu.VMEM((_CSZ, _D), jnp.float32),
            pltpu.VMEM((_CSZ, _D), jnp.float32),
            pltpu.VMEM((_CSZ, _D), jnp.float32),
            pltpu.VMEM_SHARED((_N, _D), jnp.float32),
            pltpu.SemaphoreType.DMA,
            pltpu.SemaphoreType.DMA,
            pltpu.SemaphoreType.DMA,
            pltpu.SemaphoreType.DMA,
            pltpu.SemaphoreType.DMA,
            pltpu.SemaphoreType.DMA,
        ],
    )
    def k(msg_h, dst_h, z_h, out_h, dst_v, buf0, buf1, buf2, acc,
          rsem0, rsem1, rsem2, ssem0, ssem1, ssem2):
        c = lax.axis_index("c")
        s = lax.axis_index("s")
        wid = s * 2 + c
        bufs = (buf0, buf1, buf2)
        rsems = (rsem0, rsem1, rsem2)
        ssems = (ssem0, ssem1, ssem2)

        def msg_slice(ci):
            return msg_h.at[wid, pl.ds(pl.multiple_of(ci * _CSZ, 64), _CSZ)]

        def issue_read(ci, r):
            pltpu.async_copy(msg_slice(ci), bufs[r], rsems[r])

        def wait_read(ci, r):
            pltpu.make_async_copy(msg_slice(ci), bufs[r], rsems[r]).wait()

        def issue_scat(ci, r):
            pltpu.async_copy(bufs[r], acc.at[dst_v.at[ci]], ssems[r],
                             add=True)

        def wait_scat(ci, r):
            pltpu.make_async_copy(bufs[r], acc.at[dst_v.at[ci]],
                                  ssems[r]).wait()

        _stage_rows(z_h, acc, s)
        pltpu.sync_copy(dst_h.at[wid], dst_v)
        plsc.subcore_barrier()

        issue_read(0, 0)
        issue_read(1, 1)
        for ci in range(3):
            wait_read(ci, ci)
            issue_scat(ci, ci)
            if ci >= 1:
                wait_scat(ci - 1, ci - 1)
            issue_read(ci + 2, (ci + 2) % 3)

        def body(i, _):
            for r in range(3):
                ci = 3 * i + r
                wait_read(ci, r)
                issue_scat(ci, r)

                @pl.when(ci + 2 < _CP)
                def _():
                    wait_scat(ci - 1, (r - 1) % 3)
                    issue_read(ci + 2, (r + 2) % 3)
            return 0

        lax.fori_loop(1, _CP // 3, body, 0)
        for ci in range(_CP - 3, _CP):
            wait_scat(ci, ci % 3)
        plsc.subcore_barrier()
        _stage_rows(acc, out_h.at[c], s)

    return k(msg, dst4, zeros)


def _tc_prologue(ue, it, Wu1, bu1, Wu2, bu2, Wi1, bi1, Wi2, bi2,
                 g0, b0, gb0, bb0):
    def body(ue_r, it_r, wu1, bu1r, wu2, bu2r, wi1, bi1r, wi2, bi2r,
             g0r, b0r, gb0r, bb0r, out_r):
        f32 = jnp.float32
        u = ue_r[...]
        h = jnp.maximum(jnp.dot(u, wu1[...], preferred_element_type=f32)
                        + bu1r[...], 0.0)
        u2 = jnp.dot(h, wu2[...], preferred_element_type=f32) + bu2r[...]
        t = it_r[...]
        h2 = jnp.maximum(jnp.dot(t, wi1[...], preferred_element_type=f32)
                         + bi1r[...], 0.0)
        t2 = jnp.dot(h2, wi2[...], preferred_element_type=f32) + bi2r[...]
        x = jnp.concatenate([u2, t2], axis=0)
        mu = jnp.mean(x, axis=1, keepdims=True)
        var = jnp.mean((x - mu) ** 2, axis=1, keepdims=True)
        x = (x - mu) / jnp.sqrt(var + _EPS) * g0r[...] + b0r[...]
        mu0 = jnp.mean(x, axis=0, keepdims=True)
        var0 = jnp.mean((x - mu0) ** 2, axis=0, keepdims=True)
        out_r[...] = (x - mu0) / jnp.sqrt(var0 + _EPS) * gb0r[...] + bb0r[...]

    return pl.pallas_call(
        body,
        out_shape=jax.ShapeDtypeStruct((_N, _D), jnp.float32),
    )(ue, it, Wu1, bu1, Wu2, bu2, Wi1, bi1, Wi2, bi2, g0, b0, gb0, bb0)


def _tc_layer(agg, ori, g, b, gb, bb, relu):
    def body(agg_r, ori_r, g_r, b_r, gb_r, bb_r, out_r):
        x = ori_r[...] + agg_r[0] + agg_r[1]
        mu = jnp.mean(x, axis=1, keepdims=True)
        var = jnp.mean((x - mu) ** 2, axis=1, keepdims=True)
        x = (x - mu) / jnp.sqrt(var + _EPS) * g_r[...] + b_r[...]
        mu0 = jnp.mean(x, axis=0, keepdims=True)
        var0 = jnp.mean((x - mu0) ** 2, axis=0, keepdims=True)
        x = (x - mu0) / jnp.sqrt(var0 + _EPS) * gb_r[...] + bb_r[...]
        if relu:
            x = jnp.maximum(x, 0.0)
        out_r[...] = x

    return pl.pallas_call(
        body,
        out_shape=jax.ShapeDtypeStruct((_N, _D), jnp.float32),
    )(agg, ori, g, b, gb, bb)


def kernel(edge_weight, user_table, faker_table, item_table, Wi1, bi1, Wi2,
           bi2, Wu1, bu1, Wu2, bu2, ln0_g, ln0_b, bn0_g, bn0_b, ln_g, ln_b,
           bn_g, bn_b, edge_index):
    ue = jnp.concatenate([user_table, faker_table], axis=0)
    r = lambda v: v.reshape(1, _D)

    pad = _EPAD - _E
    src4 = jnp.pad(edge_index[1], (0, pad)).astype(jnp.int32)
    src4 = src4.reshape(_NW, _CP // 2, 2 * _CSZ)
    dst4 = jnp.pad(edge_index[0], (0, pad)).astype(jnp.int32)
    dst4 = dst4.reshape(_NW, _CP, _CSZ)
    w4 = jnp.pad(edge_weight, (0, pad)).reshape(_NW, _EPW)
    zeros = jnp.zeros((_N, _D), jnp.float32)

    embs_ori = _tc_prologue(ue, item_table, Wu1, r(bu1), Wu2, r(bu2),
                            Wi1, r(bi1), Wi2, r(bi2),
                            r(ln0_g), r(ln0_b), r(bn0_g), r(bn0_b))
    x = embs_ori
    for layer in range(3):
        msg = _sc_gather_scale(x, src4, w4)
        agg = _sc_scatter_add(msg, dst4, zeros)
        x = _tc_layer(agg, embs_ori, r(ln_g[layer]), r(ln_b[layer]),
                      r(bn_g[layer]), r(bn_b[layer]), relu=layer != 2)
    return x[:_NUF], x[_NUF:]
